# baseline (speedup 1.0000x reference)
import jax
import jax.numpy as jnp
from jax.experimental import pallas as pl
from jax.experimental.pallas import tpu as pltpu


def _conv_taps(xp_ref, w_ref, oh, ow):
    _, _, C, Bt, _, _ = xp_ref.shape
    acc = None
    for dy in range(3):
        for dx in range(3):
            tap = xp_ref[dy % 2, dx % 2, :, :,
                         pl.ds(dy // 2, oh), pl.ds(dx // 2, ow)]
            wt = w_ref[:, pl.ds((dy * 3 + dx) * C, C)]
            d = jax.lax.dot_general(wt, tap.reshape(C, Bt * oh * ow),
                                    (((1,), (0,)), ((), ())),
                                    preferred_element_type=jnp.float32)
            acc = d if acc is None else acc + d
    return acc


def _conv_body(xp_ref, w_ref, b_ref, o_ref, *, oh, ow):
    Bt = xp_ref.shape[3]
    acc = _conv_taps(xp_ref, w_ref, oh, ow)
    y = jnp.maximum(acc + b_ref[...], 0.0).astype(jnp.bfloat16)
    o_ref[...] = y.reshape(w_ref.shape[0], Bt, oh, ow)


def _conv_call(xp, w, b, oh, ow, bt):
    _, _, C, B, hh, wh = xp.shape
    OC = w.shape[0]
    from functools import partial
    return pl.pallas_call(
        partial(_conv_body, oh=oh, ow=ow),
        out_shape=jax.ShapeDtypeStruct((OC, B, oh, ow), jnp.bfloat16),
        grid=(B // bt,),
        in_specs=[
            pl.BlockSpec((2, 2, C, bt, hh, wh), lambda i: (0, 0, 0, i, 0, 0)),
            pl.BlockSpec(w.shape, lambda i: (0, 0)),
            pl.BlockSpec(b.shape, lambda i: (0, 0)),
        ],
        out_specs=pl.BlockSpec((OC, bt, oh, ow), lambda i: (0, i, 0, 0)),
        compiler_params=pltpu.CompilerParams(
            dimension_semantics=("parallel",),
            vmem_limit_bytes=56 * 1024 * 1024),
    )(xp, w, b)


def _conv3_pool_body(xp_ref, w_ref, b_ref, o_ref):
    Bt = xp_ref.shape[3]
    acc = _conv_taps(xp_ref, w_ref, 12, 12)
    y = jnp.maximum(acc + b_ref[...], 0.0).astype(jnp.bfloat16)
    y = y.reshape(32, Bt, 12, 12)
    y = jnp.max(y.reshape(32, Bt, 12, 6, 2), axis=4)
    y = jnp.max(y.reshape(32, Bt, 6, 2, 6), axis=3)
    o_ref[...] = jnp.transpose(y, (1, 0, 2, 3)).reshape(Bt, 1152)


def _fc_body(x_ref, w1_ref, b1_ref, w2_ref, b2_ref, o_ref):
    h = jnp.dot(x_ref[...], w1_ref[...], preferred_element_type=jnp.float32)
    h = jnp.maximum(h + b1_ref[...], 0.0).astype(jnp.bfloat16)
    logits = jnp.dot(h, w2_ref[...],
                     preferred_element_type=jnp.float32) + b2_ref[...]
    m = jnp.max(logits, axis=-1, keepdims=True)
    e = jnp.exp(logits - m)
    o_ref[...] = e / jnp.sum(e, axis=-1, keepdims=True)


def _parity_split(y, hh, wh):
    planes = []
    for pr in range(2):
        row = []
        for pc in range(2):
            p = y[:, :, pr::2, pc::2]
            row.append(jnp.pad(p, ((0, 0), (0, 0),
                                   (0, hh - p.shape[2]), (0, wh - p.shape[3]))))
        planes.append(jnp.stack(row, axis=0))
    return jnp.stack(planes, axis=0)


def kernel(x, conv1_w, conv1_b, conv2_w, conv2_b, conv3_w, conv3_b,
           fc1_w, fc1_b, out_w, out_b):
    B = x.shape[0]
    xc = jnp.transpose(x, (1, 0, 2, 3)).astype(jnp.bfloat16)
    xp = _parity_split(xc, 52, 52)

    bt = 16 if B % 16 == 0 else B
    y1 = _conv_call(xp, conv1_w, conv1_b, 51, 51, bt)
    y1p = _parity_split(y1, 26, 26)
    y2 = _conv_call(y1p, conv2_w, conv2_b, 25, 25, bt)
    y2p = _parity_split(y2, 13, 13)
    flat = pl.pallas_call(
        _conv3_pool_body,
        out_shape=jax.ShapeDtypeStruct((B, 1152), jnp.bfloat16),
        grid=(B // bt,),
        in_specs=[
            pl.BlockSpec((2, 2, 64, bt, 13, 13),
                         lambda i: (0, 0, 0, i, 0, 0)),
            pl.BlockSpec(conv3_w.shape, lambda i: (0, 0)),
            pl.BlockSpec(conv3_b.shape, lambda i: (0, 0)),
        ],
        out_specs=pl.BlockSpec((bt, 1152), lambda i: (i, 0)),
        compiler_params=pltpu.CompilerParams(
            dimension_semantics=("parallel",),
            vmem_limit_bytes=56 * 1024 * 1024),
    )(y2p, conv3_w, conv3_b)

    bf = B // 2
    return pl.pallas_call(
        _fc_body,
        out_shape=jax.ShapeDtypeStruct((B, 10), jnp.float32),
        grid=(2,),
        in_specs=[
            pl.BlockSpec((bf, 1152), lambda i: (i, 0)),
            pl.BlockSpec(fc1_w.shape, lambda i: (0, 0)),
            pl.BlockSpec(fc1_b.shape, lambda i: (0, 0)),
            pl.BlockSpec(out_w.shape, lambda i: (0, 0)),
            pl.BlockSpec(out_b.shape, lambda i: (0, 0)),
        ],
        out_specs=pl.BlockSpec((bf, 10), lambda i: (i, 0)),
        compiler_params=pltpu.CompilerParams(
            dimension_semantics=("parallel",),
            vmem_limit_bytes=56 * 1024 * 1024),
    )(flat, fc1_w, fc1_b, out_w, out_b)


# speedup vs baseline: 2.0192x; 2.0192x over previous
import jax
import jax.numpy as jnp
from jax.experimental import pallas as pl
from jax.experimental.pallas import tpu as pltpu


def _conv_taps(xp_ref, w_ref, oh, ow):
    _, _, C, Bt, _, _ = xp_ref.shape
    acc = None
    for dy in range(3):
        for dx in range(3):
            tap = xp_ref[dy % 2, dx % 2, :, :,
                         pl.ds(dy // 2, oh), pl.ds(dx // 2, ow)]
            wt = w_ref[:, pl.ds((dy * 3 + dx) * C, C)]
            d = jax.lax.dot_general(wt, tap.reshape(C, Bt * oh * ow),
                                    (((1,), (0,)), ((), ())),
                                    preferred_element_type=jnp.float32)
            acc = d if acc is None else acc + d
    return acc


def _conv_body(xp_ref, w_ref, b_ref, o_ref, *, oh, ow):
    Bt = xp_ref.shape[3]
    acc = _conv_taps(xp_ref, w_ref, oh, ow)
    y = jnp.maximum(acc + b_ref[...], 0.0).astype(jnp.bfloat16)
    o_ref[...] = y.reshape(w_ref.shape[0], Bt, oh, ow)


def _conv_call(xp, w, b, oh, ow, bt):
    _, _, C, B, hh, wh = xp.shape
    OC = w.shape[0]
    from functools import partial
    return pl.pallas_call(
        partial(_conv_body, oh=oh, ow=ow),
        out_shape=jax.ShapeDtypeStruct((OC, B, oh, ow), jnp.bfloat16),
        grid=(B // bt,),
        in_specs=[
            pl.BlockSpec((2, 2, C, bt, hh, wh), lambda i: (0, 0, 0, i, 0, 0)),
            pl.BlockSpec(w.shape, lambda i: (0, 0)),
            pl.BlockSpec(b.shape, lambda i: (0, 0)),
        ],
        out_specs=pl.BlockSpec((OC, bt, oh, ow), lambda i: (0, i, 0, 0)),
        compiler_params=pltpu.CompilerParams(
            dimension_semantics=("parallel",),
            vmem_limit_bytes=56 * 1024 * 1024),
    )(xp, w, b)


def _conv3_pool_body(xp_ref, w_ref, b_ref, o_ref):
    Bt = xp_ref.shape[3]
    acc = _conv_taps(xp_ref, w_ref, 12, 12)
    y = jnp.maximum(acc + b_ref[...], 0.0).astype(jnp.bfloat16)
    y = y.reshape(32, Bt, 12, 12)
    y = jnp.max(y.reshape(32, Bt, 12, 6, 2), axis=4)
    y = jnp.max(y.reshape(32, Bt, 6, 2, 6), axis=3)
    o_ref[...] = jnp.transpose(y, (1, 0, 2, 3)).reshape(Bt, 1152)


def _fc_body(x_ref, w1_ref, b1_ref, w2_ref, b2_ref, o_ref):
    h = jnp.dot(x_ref[...], w1_ref[...], preferred_element_type=jnp.float32)
    h = jnp.maximum(h + b1_ref[...], 0.0).astype(jnp.bfloat16)
    logits = jnp.dot(h, w2_ref[...],
                     preferred_element_type=jnp.float32) + b2_ref[...]
    m = jnp.max(logits, axis=-1, keepdims=True)
    e = jnp.exp(logits - m)
    o_ref[...] = e / jnp.sum(e, axis=-1, keepdims=True)


def _parity_split(y, hh, wh):
    C, B, H, W = y.shape
    yp = jnp.pad(y, ((0, 0), (0, 0), (0, 2 * hh - H), (0, 2 * wh - W)))
    yp = yp.reshape(C, B, hh, 2, wh, 2)
    return jnp.transpose(yp, (3, 5, 0, 1, 2, 4))


def kernel(x, conv1_w, conv1_b, conv2_w, conv2_b, conv3_w, conv3_b,
           fc1_w, fc1_b, out_w, out_b):
    B = x.shape[0]
    xc = jnp.transpose(x, (1, 0, 2, 3)).astype(jnp.bfloat16)
    xp = _parity_split(xc, 52, 52)

    bt = 16 if B % 16 == 0 else B
    y1 = _conv_call(xp, conv1_w, conv1_b, 51, 51, bt)
    y1p = _parity_split(y1, 26, 26)
    y2 = _conv_call(y1p, conv2_w, conv2_b, 25, 25, bt)
    y2p = _parity_split(y2, 13, 13)
    flat = pl.pallas_call(
        _conv3_pool_body,
        out_shape=jax.ShapeDtypeStruct((B, 1152), jnp.bfloat16),
        grid=(B // bt,),
        in_specs=[
            pl.BlockSpec((2, 2, 64, bt, 13, 13),
                         lambda i: (0, 0, 0, i, 0, 0)),
            pl.BlockSpec(conv3_w.shape, lambda i: (0, 0)),
            pl.BlockSpec(conv3_b.shape, lambda i: (0, 0)),
        ],
        out_specs=pl.BlockSpec((bt, 1152), lambda i: (i, 0)),
        compiler_params=pltpu.CompilerParams(
            dimension_semantics=("parallel",),
            vmem_limit_bytes=56 * 1024 * 1024),
    )(y2p, conv3_w, conv3_b)

    bf = B // 2
    return pl.pallas_call(
        _fc_body,
        out_shape=jax.ShapeDtypeStruct((B, 10), jnp.float32),
        grid=(2,),
        in_specs=[
            pl.BlockSpec((bf, 1152), lambda i: (i, 0)),
            pl.BlockSpec(fc1_w.shape, lambda i: (0, 0)),
            pl.BlockSpec(fc1_b.shape, lambda i: (0, 0)),
            pl.BlockSpec(out_w.shape, lambda i: (0, 0)),
            pl.BlockSpec(out_b.shape, lambda i: (0, 0)),
        ],
        out_specs=pl.BlockSpec((bf, 10), lambda i: (i, 0)),
        compiler_params=pltpu.CompilerParams(
            dimension_semantics=("parallel",),
            vmem_limit_bytes=56 * 1024 * 1024),
    )(flat, fc1_w, fc1_b, out_w, out_b)


# speedup vs baseline: 3.3629x; 1.6654x over previous
import jax
import jax.numpy as jnp
from jax.experimental import pallas as pl
from jax.experimental.pallas import tpu as pltpu


def _conv_taps(xp_ref, w_ref, oh, ow):
    _, _, C, Bt, _, _ = xp_ref.shape
    acc = None
    for dy in range(3):
        for dx in range(3):
            tap = xp_ref[dy % 2, dx % 2, :, :,
                         pl.ds(dy // 2, oh), pl.ds(dx // 2, ow)]
            wt = w_ref[:, pl.ds((dy * 3 + dx) * C, C)]
            d = jax.lax.dot_general(wt, tap.reshape(C, Bt * oh * ow),
                                    (((1,), (0,)), ((), ())),
                                    preferred_element_type=jnp.float32)
            acc = d if acc is None else acc + d
    return acc


def _conv_body(xp_ref, w_ref, b_ref, o_ref, *, oh, ow):
    Bt = xp_ref.shape[3]
    acc = _conv_taps(xp_ref, w_ref, oh, ow)
    y = jnp.maximum(acc + b_ref[...], 0.0).astype(jnp.bfloat16)
    o_ref[...] = y.reshape(w_ref.shape[0], Bt, oh, ow).astype(o_ref.dtype)


def _conv2_body(x_ref, w_ref, b_ref, o_ref, t_ref):
    C, Bt, _, _ = x_ref.shape
    acc = None
    for dy in range(3):
        slab = x_ref[:, :, pl.ds(dy, 25, 2), :]
        t_ref[...] = jnp.swapaxes(slab, 2, 3)
        for dx in range(3):
            tap = t_ref[:, :, pl.ds(dx, 25, 2), :]
            wt = w_ref[:, pl.ds((dy * 3 + dx) * C, C)]
            d = jax.lax.dot_general(
                wt, tap.astype(jnp.bfloat16).reshape(C, Bt * 625),
                (((1,), (0,)), ((), ())),
                preferred_element_type=jnp.float32)
            acc = d if acc is None else acc + d
    y = jnp.maximum(acc + b_ref[...], 0.0).astype(jnp.bfloat16)
    y = jnp.swapaxes(y.reshape(64, Bt, 25, 25), 2, 3)
    o_ref[...] = y


def _conv_call(xp, w, b, oh, ow, bt, out_dtype=jnp.bfloat16):
    _, _, C, B, hh, wh = xp.shape
    OC = w.shape[0]
    from functools import partial
    return pl.pallas_call(
        partial(_conv_body, oh=oh, ow=ow),
        out_shape=jax.ShapeDtypeStruct((OC, B, oh, ow), out_dtype),
        grid=(B // bt,),
        in_specs=[
            pl.BlockSpec((2, 2, C, bt, hh, wh), lambda i: (0, 0, 0, i, 0, 0)),
            pl.BlockSpec(w.shape, lambda i: (0, 0)),
            pl.BlockSpec(b.shape, lambda i: (0, 0)),
        ],
        out_specs=pl.BlockSpec((OC, bt, oh, ow), lambda i: (0, i, 0, 0)),
        compiler_params=pltpu.CompilerParams(
            dimension_semantics=("parallel",),
            vmem_limit_bytes=56 * 1024 * 1024),
    )(xp, w, b)


def _conv3_pool_body(xp_ref, w_ref, b_ref, o_ref):
    Bt = xp_ref.shape[3]
    acc = _conv_taps(xp_ref, w_ref, 12, 12)
    y = jnp.maximum(acc + b_ref[...], 0.0).astype(jnp.bfloat16)
    y = y.reshape(32, Bt, 12, 12)
    y = jnp.max(y.reshape(32, Bt, 12, 6, 2), axis=4)
    y = jnp.max(y.reshape(32, Bt, 6, 2, 6), axis=3)
    o_ref[...] = jnp.transpose(y, (1, 0, 2, 3)).reshape(Bt, 1152)


def _fc_body(x_ref, w1_ref, b1_ref, w2_ref, b2_ref, o_ref):
    h = jnp.dot(x_ref[...], w1_ref[...], preferred_element_type=jnp.float32)
    h = jnp.maximum(h + b1_ref[...], 0.0).astype(jnp.bfloat16)
    logits = jnp.dot(h, w2_ref[...],
                     preferred_element_type=jnp.float32) + b2_ref[...]
    m = jnp.max(logits, axis=-1, keepdims=True)
    e = jnp.exp(logits - m)
    o_ref[...] = e / jnp.sum(e, axis=-1, keepdims=True)


def _parity_split(y, hh, wh):
    C, B, H, W = y.shape
    yp = jnp.pad(y, ((0, 0), (0, 0), (0, 2 * hh - H), (0, 2 * wh - W)))
    yp = yp.reshape(C, B, hh, 2, wh, 2)
    return jnp.transpose(yp, (3, 5, 0, 1, 2, 4))


def kernel(x, conv1_w, conv1_b, conv2_w, conv2_b, conv3_w, conv3_b,
           fc1_w, fc1_b, out_w, out_b):
    B = x.shape[0]
    xc = jnp.transpose(x, (1, 0, 2, 3)).astype(jnp.bfloat16)
    xp = _parity_split(xc, 52, 52)

    bt = 16 if B % 16 == 0 else B
    bt2 = 8 if B % 8 == 0 else B
    y1 = _conv_call(xp, conv1_w, conv1_b, 51, 51, bt2,
                    jnp.float32)
    y2 = pl.pallas_call(
        _conv2_body,
        out_shape=jax.ShapeDtypeStruct((64, B, 25, 25), jnp.bfloat16),
        grid=(B // bt2,),
        in_specs=[
            pl.BlockSpec((32, bt2, 51, 51), lambda i: (0, i, 0, 0)),
            pl.BlockSpec(conv2_w.shape, lambda i: (0, 0)),
            pl.BlockSpec(conv2_b.shape, lambda i: (0, 0)),
        ],
        out_specs=pl.BlockSpec((64, bt2, 25, 25), lambda i: (0, i, 0, 0)),
        scratch_shapes=[pltpu.VMEM((32, bt2, 51, 25), jnp.float32)],
        compiler_params=pltpu.CompilerParams(
            dimension_semantics=("parallel",),
            vmem_limit_bytes=56 * 1024 * 1024),
    )(y1, conv2_w, conv2_b)
    y2p = _parity_split(y2, 13, 13)
    flat = pl.pallas_call(
        _conv3_pool_body,
        out_shape=jax.ShapeDtypeStruct((B, 1152), jnp.bfloat16),
        grid=(B // bt,),
        in_specs=[
            pl.BlockSpec((2, 2, 64, bt, 13, 13),
                         lambda i: (0, 0, 0, i, 0, 0)),
            pl.BlockSpec(conv3_w.shape, lambda i: (0, 0)),
            pl.BlockSpec(conv3_b.shape, lambda i: (0, 0)),
        ],
        out_specs=pl.BlockSpec((bt, 1152), lambda i: (i, 0)),
        compiler_params=pltpu.CompilerParams(
            dimension_semantics=("parallel",),
            vmem_limit_bytes=56 * 1024 * 1024),
    )(y2p, conv3_w, conv3_b)

    bf = B // 2
    return pl.pallas_call(
        _fc_body,
        out_shape=jax.ShapeDtypeStruct((B, 10), jnp.float32),
        grid=(2,),
        in_specs=[
            pl.BlockSpec((bf, 1152), lambda i: (i, 0)),
            pl.BlockSpec(fc1_w.shape, lambda i: (0, 0)),
            pl.BlockSpec(fc1_b.shape, lambda i: (0, 0)),
            pl.BlockSpec(out_w.shape, lambda i: (0, 0)),
            pl.BlockSpec(out_b.shape, lambda i: (0, 0)),
        ],
        out_specs=pl.BlockSpec((bf, 10), lambda i: (i, 0)),
        compiler_params=pltpu.CompilerParams(
            dimension_semantics=("parallel",),
            vmem_limit_bytes=56 * 1024 * 1024),
    )(flat, fc1_w, fc1_b, out_w, out_b)


# speedup vs baseline: 3.8391x; 1.1416x over previous
import jax
import jax.numpy as jnp
from jax.experimental import pallas as pl
from jax.experimental.pallas import tpu as pltpu


def _conv_taps(xp_ref, w_ref, oh, ow):
    _, _, C, Bt, _, _ = xp_ref.shape
    acc = None
    for dy in range(3):
        for dx in range(3):
            tap = xp_ref[dy % 2, dx % 2, :, :,
                         pl.ds(dy // 2, oh), pl.ds(dx // 2, ow)]
            wt = w_ref[:, pl.ds((dy * 3 + dx) * C, C)]
            d = jax.lax.dot_general(wt, tap.reshape(C, Bt * oh * ow),
                                    (((1,), (0,)), ((), ())),
                                    preferred_element_type=jnp.float32)
            acc = d if acc is None else acc + d
    return acc


def _conv_body(xp_ref, w_ref, b_ref, o_ref, *, oh, ow):
    Bt = xp_ref.shape[3]
    acc = _conv_taps(xp_ref, w_ref, oh, ow)
    y = jnp.maximum(acc + b_ref[...], 0.0).astype(jnp.bfloat16)
    o_ref[...] = y.reshape(w_ref.shape[0], Bt, oh, ow).astype(o_ref.dtype)


def _conv1_body(x_ref, w_ref, b_ref, o_ref, t_ref):
    Bt = x_ref.shape[0]
    acc = None
    for dy in range(3):
        slab = x_ref[:, :, pl.ds(dy, 51, 2), :]
        ch = jnp.transpose(slab, (1, 0, 2, 3))
        t_ref[...] = jnp.swapaxes(ch, 2, 3)
        for dx in range(3):
            tap = t_ref[:, :, pl.ds(dx, 51, 2), :]
            wt = w_ref[:, pl.ds((dy * 3 + dx) * 3, 3)]
            d = jax.lax.dot_general(
                wt, tap.astype(jnp.bfloat16).reshape(3, Bt * 51 * 51),
                (((1,), (0,)), ((), ())),
                preferred_element_type=jnp.float32)
            acc = d if acc is None else acc + d
    y = jnp.maximum(acc + b_ref[...], 0.0).astype(jnp.bfloat16)
    o_ref[...] = y.reshape(32, Bt, 51, 51).astype(o_ref.dtype)


def _conv2_body(x_ref, w_ref, b_ref, o_ref, t_ref):
    C, Bt, _, _ = x_ref.shape
    acc = None
    for dx in range(3):
        slab = x_ref[:, :, pl.ds(dx, 25, 2), :]
        t_ref[...] = jnp.swapaxes(slab, 2, 3)
        for dy in range(3):
            tap = t_ref[:, :, pl.ds(dy, 25, 2), :]
            wt = w_ref[:, pl.ds((dy * 3 + dx) * C, C)]
            d = jax.lax.dot_general(
                wt, tap.astype(jnp.bfloat16).reshape(C, Bt * 625),
                (((1,), (0,)), ((), ())),
                preferred_element_type=jnp.float32)
            acc = d if acc is None else acc + d
    y = jnp.maximum(acc + b_ref[...], 0.0).astype(jnp.bfloat16)
    o_ref[...] = y.reshape(64, Bt, 25, 25)


def _conv_call(xp, w, b, oh, ow, bt, out_dtype=jnp.bfloat16):
    _, _, C, B, hh, wh = xp.shape
    OC = w.shape[0]
    from functools import partial
    return pl.pallas_call(
        partial(_conv_body, oh=oh, ow=ow),
        out_shape=jax.ShapeDtypeStruct((OC, B, oh, ow), out_dtype),
        grid=(B // bt,),
        in_specs=[
            pl.BlockSpec((2, 2, C, bt, hh, wh), lambda i: (0, 0, 0, i, 0, 0)),
            pl.BlockSpec(w.shape, lambda i: (0, 0)),
            pl.BlockSpec(b.shape, lambda i: (0, 0)),
        ],
        out_specs=pl.BlockSpec((OC, bt, oh, ow), lambda i: (0, i, 0, 0)),
        compiler_params=pltpu.CompilerParams(
            dimension_semantics=("parallel",),
            vmem_limit_bytes=56 * 1024 * 1024),
    )(xp, w, b)


def _conv3_pool_body(xp_ref, w_ref, b_ref, o_ref):
    Bt = xp_ref.shape[3]
    acc = _conv_taps(xp_ref, w_ref, 12, 12)
    y = jnp.maximum(acc + b_ref[...], 0.0).astype(jnp.bfloat16)
    y = y.reshape(32, Bt, 12, 12)
    y = jnp.max(y.reshape(32, Bt, 12, 6, 2), axis=4)
    y = jnp.max(y.reshape(32, Bt, 6, 2, 6), axis=3)
    o_ref[...] = jnp.transpose(y, (1, 0, 2, 3)).reshape(Bt, 1152)


def _fc_body(x_ref, w1_ref, b1_ref, w2_ref, b2_ref, o_ref):
    h = jnp.dot(x_ref[...], w1_ref[...], preferred_element_type=jnp.float32)
    h = jnp.maximum(h + b1_ref[...], 0.0).astype(jnp.bfloat16)
    logits = jnp.dot(h, w2_ref[...],
                     preferred_element_type=jnp.float32) + b2_ref[...]
    m = jnp.max(logits, axis=-1, keepdims=True)
    e = jnp.exp(logits - m)
    o_ref[...] = e / jnp.sum(e, axis=-1, keepdims=True)


def _parity_split(y, hh, wh):
    C, B, H, W = y.shape
    yp = jnp.pad(y, ((0, 0), (0, 0), (0, 2 * hh - H), (0, 2 * wh - W)))
    yp = yp.reshape(C, B, hh, 2, wh, 2)
    return jnp.transpose(yp, (3, 5, 0, 1, 2, 4))


def kernel(x, conv1_w, conv1_b, conv2_w, conv2_b, conv3_w, conv3_b,
           fc1_w, fc1_b, out_w, out_b):
    B = x.shape[0]
    bt = 16 if B % 16 == 0 else B
    bt2 = 8 if B % 8 == 0 else B
    y1 = pl.pallas_call(
        _conv1_body,
        out_shape=jax.ShapeDtypeStruct((32, B, 51, 51), jnp.float32),
        grid=(B // bt2,),
        in_specs=[
            pl.BlockSpec((bt2, 3, 104, 104), lambda i: (i, 0, 0, 0)),
            pl.BlockSpec(conv1_w.shape, lambda i: (0, 0)),
            pl.BlockSpec(conv1_b.shape, lambda i: (0, 0)),
        ],
        out_specs=pl.BlockSpec((32, bt2, 51, 51), lambda i: (0, i, 0, 0)),
        scratch_shapes=[pltpu.VMEM((3, bt2, 104, 51), jnp.float32)],
        compiler_params=pltpu.CompilerParams(
            dimension_semantics=("parallel",),
            vmem_limit_bytes=56 * 1024 * 1024),
    )(x, conv1_w, conv1_b)
    y2 = pl.pallas_call(
        _conv2_body,
        out_shape=jax.ShapeDtypeStruct((64, B, 25, 25), jnp.bfloat16),
        grid=(B // bt2,),
        in_specs=[
            pl.BlockSpec((32, bt2, 51, 51), lambda i: (0, i, 0, 0)),
            pl.BlockSpec(conv2_w.shape, lambda i: (0, 0)),
            pl.BlockSpec(conv2_b.shape, lambda i: (0, 0)),
        ],
        out_specs=pl.BlockSpec((64, bt2, 25, 25), lambda i: (0, i, 0, 0)),
        scratch_shapes=[pltpu.VMEM((32, bt2, 51, 25), jnp.float32)],
        compiler_params=pltpu.CompilerParams(
            dimension_semantics=("parallel",),
            vmem_limit_bytes=56 * 1024 * 1024),
    )(y1, conv2_w, conv2_b)
    y2p = _parity_split(y2, 13, 13)
    flat = pl.pallas_call(
        _conv3_pool_body,
        out_shape=jax.ShapeDtypeStruct((B, 1152), jnp.bfloat16),
        grid=(B // bt,),
        in_specs=[
            pl.BlockSpec((2, 2, 64, bt, 13, 13),
                         lambda i: (0, 0, 0, i, 0, 0)),
            pl.BlockSpec(conv3_w.shape, lambda i: (0, 0)),
            pl.BlockSpec(conv3_b.shape, lambda i: (0, 0)),
        ],
        out_specs=pl.BlockSpec((bt, 1152), lambda i: (i, 0)),
        compiler_params=pltpu.CompilerParams(
            dimension_semantics=("parallel",),
            vmem_limit_bytes=56 * 1024 * 1024),
    )(y2p, conv3_w, conv3_b)

    bf = B // 2
    return pl.pallas_call(
        _fc_body,
        out_shape=jax.ShapeDtypeStruct((B, 10), jnp.float32),
        grid=(2,),
        in_specs=[
            pl.BlockSpec((bf, 1152), lambda i: (i, 0)),
            pl.BlockSpec(fc1_w.shape, lambda i: (0, 0)),
            pl.BlockSpec(fc1_b.shape, lambda i: (0, 0)),
            pl.BlockSpec(out_w.shape, lambda i: (0, 0)),
            pl.BlockSpec(out_b.shape, lambda i: (0, 0)),
        ],
        out_specs=pl.BlockSpec((bf, 10), lambda i: (i, 0)),
        compiler_params=pltpu.CompilerParams(
            dimension_semantics=("parallel",),
            vmem_limit_bytes=56 * 1024 * 1024),
    )(flat, fc1_w, fc1_b, out_w, out_b)


# speedup vs baseline: 20.8845x; 5.4400x over previous
import jax
import jax.numpy as jnp
from jax.experimental import pallas as pl
from jax.experimental.pallas import tpu as pltpu


def _conv_taps(xp_ref, w_ref, oh, ow):
    _, _, C, Bt, _, _ = xp_ref.shape
    acc = None
    for dy in range(3):
        for dx in range(3):
            tap = xp_ref[dy % 2, dx % 2, :, :,
                         pl.ds(dy // 2, oh), pl.ds(dx // 2, ow)]
            wt = w_ref[:, pl.ds((dy * 3 + dx) * C, C)]
            d = jax.lax.dot_general(wt, tap.reshape(C, Bt * oh * ow),
                                    (((1,), (0,)), ((), ())),
                                    preferred_element_type=jnp.float32)
            acc = d if acc is None else acc + d
    return acc


def _conv_body(xp_ref, w_ref, b_ref, o_ref, *, oh, ow):
    Bt = xp_ref.shape[3]
    acc = _conv_taps(xp_ref, w_ref, oh, ow)
    y = jnp.maximum(acc + b_ref[...], 0.0).astype(jnp.bfloat16)
    o_ref[...] = y.reshape(w_ref.shape[0], Bt, oh, ow).astype(o_ref.dtype)


def _conv1_body(x_ref, w_ref, b_ref, o_ref, t_ref):
    Bt = x_ref.shape[0]
    acc = None
    for dy in range(3):
        slab = x_ref[:, :, pl.ds(dy, 51, 2), :]
        ch = jnp.transpose(slab, (1, 0, 2, 3))
        t_ref[...] = jnp.swapaxes(ch, 2, 3)
        for dx in range(3):
            tap = t_ref[:, :, pl.ds(dx, 51, 2), :]
            wt = w_ref[:, pl.ds((dy * 3 + dx) * 3, 3)]
            d = jax.lax.dot_general(
                wt, tap.astype(jnp.bfloat16).reshape(3, Bt * 51 * 51),
                (((1,), (0,)), ((), ())),
                preferred_element_type=jnp.float32)
            acc = d if acc is None else acc + d
    y = jnp.maximum(acc + b_ref[...], 0.0).astype(jnp.bfloat16)
    o_ref[...] = y.reshape(32, Bt, 51, 51).astype(o_ref.dtype)


def _conv2_body(x_ref, w_ref, b_ref, o_ref, t_ref):
    C, Bt, _, _ = x_ref.shape
    acc = None
    for dx in range(3):
        slab = x_ref[:, :, pl.ds(dx, 25, 2), :]
        t_ref[...] = jnp.swapaxes(slab, 2, 3)
        for dy in range(3):
            tap = t_ref[:, :, pl.ds(dy, 25, 2), :]
            wt = w_ref[:, pl.ds((dy * 3 + dx) * C, C)]
            d = jax.lax.dot_general(
                wt, tap.astype(jnp.bfloat16).reshape(C, Bt * 625),
                (((1,), (0,)), ((), ())),
                preferred_element_type=jnp.float32)
            acc = d if acc is None else acc + d
    y = jnp.maximum(acc + b_ref[...], 0.0).astype(jnp.bfloat16)
    o_ref[...] = y.reshape(64, Bt, 25, 25).astype(o_ref.dtype)


def _conv3_pool_fc_in_body(x_ref, w_ref, b_ref, o_ref, t_ref):
    C, Bt, _, _ = x_ref.shape
    acc = None
    for dy in range(3):
        slab = x_ref[:, :, pl.ds(dy, 12, 2), :]
        t_ref[...] = jnp.swapaxes(slab, 2, 3)
        for dx in range(3):
            tap = t_ref[:, :, pl.ds(dx, 12, 2), :]
            wt = w_ref[:, pl.ds((dy * 3 + dx) * C, C)]
            d = jax.lax.dot_general(
                wt, tap.astype(jnp.bfloat16).reshape(C, Bt * 144),
                (((1,), (0,)), ((), ())),
                preferred_element_type=jnp.float32)
            acc = d if acc is None else acc + d
    y = jnp.maximum(acc + b_ref[...], 0.0).astype(jnp.bfloat16)
    y = y.reshape(32, Bt, 12, 12)
    y = jnp.max(y.reshape(32, Bt, 12, 6, 2), axis=4)
    y = jnp.max(y.reshape(32, Bt, 6, 2, 6), axis=3)
    y = jnp.swapaxes(y, 2, 3)
    o_ref[...] = jnp.transpose(y, (1, 0, 2, 3)).reshape(Bt, 1152)


def _conv_call(xp, w, b, oh, ow, bt, out_dtype=jnp.bfloat16):
    _, _, C, B, hh, wh = xp.shape
    OC = w.shape[0]
    from functools import partial
    return pl.pallas_call(
        partial(_conv_body, oh=oh, ow=ow),
        out_shape=jax.ShapeDtypeStruct((OC, B, oh, ow), out_dtype),
        grid=(B // bt,),
        in_specs=[
            pl.BlockSpec((2, 2, C, bt, hh, wh), lambda i: (0, 0, 0, i, 0, 0)),
            pl.BlockSpec(w.shape, lambda i: (0, 0)),
            pl.BlockSpec(b.shape, lambda i: (0, 0)),
        ],
        out_specs=pl.BlockSpec((OC, bt, oh, ow), lambda i: (0, i, 0, 0)),
        compiler_params=pltpu.CompilerParams(
            dimension_semantics=("parallel",),
            vmem_limit_bytes=56 * 1024 * 1024),
    )(xp, w, b)


def _conv3_pool_body(xp_ref, w_ref, b_ref, o_ref):
    Bt = xp_ref.shape[3]
    acc = _conv_taps(xp_ref, w_ref, 12, 12)
    y = jnp.maximum(acc + b_ref[...], 0.0).astype(jnp.bfloat16)
    y = y.reshape(32, Bt, 12, 12)
    y = jnp.max(y.reshape(32, Bt, 12, 6, 2), axis=4)
    y = jnp.max(y.reshape(32, Bt, 6, 2, 6), axis=3)
    o_ref[...] = jnp.transpose(y, (1, 0, 2, 3)).reshape(Bt, 1152)


def _fc_body(x_ref, w1_ref, b1_ref, w2_ref, b2_ref, o_ref):
    h = jnp.dot(x_ref[...], w1_ref[...], preferred_element_type=jnp.float32)
    h = jnp.maximum(h + b1_ref[...], 0.0).astype(jnp.bfloat16)
    logits = jnp.dot(h, w2_ref[...],
                     preferred_element_type=jnp.float32) + b2_ref[...]
    m = jnp.max(logits, axis=-1, keepdims=True)
    e = jnp.exp(logits - m)
    o_ref[...] = e / jnp.sum(e, axis=-1, keepdims=True)


def _parity_split(y, hh, wh):
    C, B, H, W = y.shape
    yp = jnp.pad(y, ((0, 0), (0, 0), (0, 2 * hh - H), (0, 2 * wh - W)))
    yp = yp.reshape(C, B, hh, 2, wh, 2)
    return jnp.transpose(yp, (3, 5, 0, 1, 2, 4))


def kernel(x, conv1_w, conv1_b, conv2_w, conv2_b, conv3_w, conv3_b,
           fc1_w, fc1_b, out_w, out_b):
    B = x.shape[0]
    bt = 16 if B % 16 == 0 else B
    bt2 = 8 if B % 8 == 0 else B
    bt4 = 4 if B % 4 == 0 else B
    y1 = pl.pallas_call(
        _conv1_body,
        out_shape=jax.ShapeDtypeStruct((32, B, 51, 51), jnp.float32),
        grid=(B // bt2,),
        in_specs=[
            pl.BlockSpec((bt2, 3, 104, 104), lambda i: (i, 0, 0, 0)),
            pl.BlockSpec(conv1_w.shape, lambda i: (0, 0)),
            pl.BlockSpec(conv1_b.shape, lambda i: (0, 0)),
        ],
        out_specs=pl.BlockSpec((32, bt2, 51, 51), lambda i: (0, i, 0, 0)),
        scratch_shapes=[pltpu.VMEM((3, bt2, 104, 51), jnp.float32)],
        compiler_params=pltpu.CompilerParams(
            dimension_semantics=("parallel",),
            vmem_limit_bytes=56 * 1024 * 1024),
    )(x, conv1_w, conv1_b)
    y2 = pl.pallas_call(
        _conv2_body,
        out_shape=jax.ShapeDtypeStruct((64, B, 25, 25), jnp.float32),
        grid=(B // bt4,),
        in_specs=[
            pl.BlockSpec((32, bt4, 51, 51), lambda i: (0, i, 0, 0)),
            pl.BlockSpec(conv2_w.shape, lambda i: (0, 0)),
            pl.BlockSpec(conv2_b.shape, lambda i: (0, 0)),
        ],
        out_specs=pl.BlockSpec((64, bt4, 25, 25), lambda i: (0, i, 0, 0)),
        scratch_shapes=[pltpu.VMEM((32, bt4, 51, 25), jnp.float32)],
        compiler_params=pltpu.CompilerParams(
            dimension_semantics=("parallel",),
            vmem_limit_bytes=56 * 1024 * 1024),
    )(y1, conv2_w, conv2_b)
    flat = pl.pallas_call(
        _conv3_pool_fc_in_body,
        out_shape=jax.ShapeDtypeStruct((B, 1152), jnp.bfloat16),
        grid=(B // bt2,),
        in_specs=[
            pl.BlockSpec((64, bt2, 25, 25), lambda i: (0, i, 0, 0)),
            pl.BlockSpec(conv3_w.shape, lambda i: (0, 0)),
            pl.BlockSpec(conv3_b.shape, lambda i: (0, 0)),
        ],
        out_specs=pl.BlockSpec((bt2, 1152), lambda i: (i, 0)),
        scratch_shapes=[pltpu.VMEM((64, bt2, 25, 12), jnp.float32)],
        compiler_params=pltpu.CompilerParams(
            dimension_semantics=("parallel",),
            vmem_limit_bytes=56 * 1024 * 1024),
    )(y2, conv3_w, conv3_b)

    bf = B // 2
    return pl.pallas_call(
        _fc_body,
        out_shape=jax.ShapeDtypeStruct((B, 10), jnp.float32),
        grid=(2,),
        in_specs=[
            pl.BlockSpec((bf, 1152), lambda i: (i, 0)),
            pl.BlockSpec(fc1_w.shape, lambda i: (0, 0)),
            pl.BlockSpec(fc1_b.shape, lambda i: (0, 0)),
            pl.BlockSpec(out_w.shape, lambda i: (0, 0)),
            pl.BlockSpec(out_b.shape, lambda i: (0, 0)),
        ],
        out_specs=pl.BlockSpec((bf, 10), lambda i: (i, 0)),
        compiler_params=pltpu.CompilerParams(
            dimension_semantics=("parallel",),
            vmem_limit_bytes=56 * 1024 * 1024),
    )(flat, fc1_w, fc1_b, out_w, out_b)
